# baseline (speedup 1.0000x reference)
import functools

import jax
import jax.numpy as jnp
from jax import lax
from jax.experimental import pallas as pl
from jax.experimental.pallas import tpu as pltpu
from jax.experimental.pallas import tpu_sc as plsc

B_STATIC = 4


def kernel(B, T, pe_weight):
    max_len, d_model = pe_weight.shape
    info = plsc.get_sparse_core_info()
    nc, ns = info.num_cores, info.num_subcores
    nw = nc * ns
    rows = max_len // nw

    mesh = plsc.VectorSubcoreMesh(core_axis_name="c", subcore_axis_name="s")

    @functools.partial(
        pl.kernel,
        mesh=mesh,
        out_type=jax.ShapeDtypeStruct((B_STATIC, max_len, d_model), pe_weight.dtype),
        scratch_types=[pltpu.SemaphoreType.DMA],
    )
    def sc_copy(table_hbm, out_hbm, sem):
        wid = lax.axis_index("s") * nc + lax.axis_index("c")
        base = wid * rows
        src = table_hbm.at[pl.ds(base, rows)]
        copies = [
            pltpu.async_copy(src, out_hbm.at[b, pl.ds(base, rows)], sem)
            for b in range(B_STATIC)
        ]
        for c in copies:
            c.wait()

    return sc_copy(pe_weight)


# speedup vs baseline: 55.1935x; 55.1935x over previous
import functools

import jax
import jax.numpy as jnp
from jax import lax
from jax.experimental import pallas as pl
from jax.experimental.pallas import tpu as pltpu
from jax.experimental.pallas import tpu_sc as plsc

B_STATIC = 4
CHUNK = 64


def kernel(B, T, pe_weight):
    max_len, d_model = pe_weight.shape
    info = plsc.get_sparse_core_info()
    nc, ns = info.num_cores, info.num_subcores
    nw = nc * ns
    rows = max_len // nw
    nchunks = rows // CHUNK

    mesh = plsc.VectorSubcoreMesh(core_axis_name="c", subcore_axis_name="s")

    @functools.partial(
        pl.kernel,
        mesh=mesh,
        out_type=jax.ShapeDtypeStruct((B_STATIC, max_len, d_model), pe_weight.dtype),
        scratch_types=[
            pltpu.VMEM((CHUNK, d_model), pe_weight.dtype),
            pltpu.SemaphoreType.DMA,
        ],
    )
    def sc_copy(table_hbm, out_hbm, buf, sem):
        wid = lax.axis_index("s") * nc + lax.axis_index("c")
        base = wid * rows

        def body(i, carry):
            start = base + i * CHUNK
            pltpu.sync_copy(table_hbm.at[pl.ds(start, CHUNK)], buf)
            copies = [
                pltpu.async_copy(buf, out_hbm.at[b, pl.ds(start, CHUNK)], sem)
                for b in range(B_STATIC)
            ]
            for c in copies:
                c.wait()
            return carry

        lax.fori_loop(0, nchunks, body, 0)

    return sc_copy(pe_weight)
